# baseline (speedup 1.0000x reference)
import functools

import jax
import jax.numpy as jnp
from jax.experimental import pallas as pl
from jax.experimental.pallas import tpu as pltpu

B, L, N = 1024, 20, 1000
EMB, U = 64, 128
ALPHA = 0.5
NP = 1024
EP = 128
BE = 256
BB = 256


def _enc_body(x_ref, ahi_ref, alo_ref, cb_ref, wrhi_ref, wrlo_ref, wenc_ref,
              benc_ref, emb_ref):
    thr = jnp.abs(cb_ref[0, 0])
    for t in range(L):
        x = x_ref[:, t, :].astype(jnp.bfloat16)
        xa = jnp.dot(x, ahi_ref[...], preferred_element_type=jnp.float32)
        xa += jnp.dot(x, alo_ref[...], preferred_element_type=jnp.float32)
        ga = jax.nn.relu(xa - thr)
        emb = jnp.dot(x, wrhi_ref[...], preferred_element_type=jnp.float32)
        emb += jnp.dot(x, wrlo_ref[...], preferred_element_type=jnp.float32)
        emb += jnp.dot(ga, wenc_ref[...], preferred_element_type=jnp.float32)
        emb_ref[:, t * EP:(t + 1) * EP] = jax.nn.relu(emb + benc_ref[...])


def _lstm_body(emb_ref, lenf_ref, a_ref, ib_ref, wx_ref, wh_ref, bl_ref,
               wH_ref, out_ref, h_ref, c_ref, hT_ref):
    t = pl.program_id(1)

    @pl.when(t == 0)
    def _init():
        h_ref[...] = jnp.zeros_like(h_ref)
        c_ref[...] = jnp.zeros_like(c_ref)
        hT_ref[...] = jnp.zeros_like(hT_ref)

    z = jnp.dot(emb_ref[...], wx_ref[...], preferred_element_type=jnp.float32)
    z += jnp.dot(h_ref[...], wh_ref[...], preferred_element_type=jnp.float32)
    z += bl_ref[...]
    i = jax.nn.sigmoid(z[:, :U])
    f = jax.nn.sigmoid(z[:, U:2 * U])
    g = jnp.tanh(z[:, 2 * U:3 * U])
    o = jax.nn.sigmoid(z[:, 3 * U:])
    c = f * c_ref[...] + i * g
    h = o * jnp.tanh(c)
    c_ref[...] = c
    h_ref[...] = h
    mask = lenf_ref[...] == t.astype(jnp.float32)
    hT_ref[...] = jnp.where(mask, h, hT_ref[...])

    @pl.when(t == L - 1)
    def _head():
        hT = hT_ref[...]
        p = jax.nn.sigmoid(
            jnp.dot(hT, wH_ref[...], preferred_element_type=jnp.float32))
        pa = jnp.dot(p, a_ref[...], preferred_element_type=jnp.float32)
        r = jax.nn.relu(ib_ref[...])
        logits = (1.0 - ALPHA) * p + ALPHA * (p * r + jax.nn.relu(pa))
        out_ref[...] = jax.nn.sigmoid(logits)[:, :N]


@functools.partial(jax.jit, static_argnames=("interpret",))
def kernel(bseq, bseq_length, A, I_B, C_B, W_enc, b_enc, Wx, Wh, b_lstm, W_H,
           interpret=False):
    pad = NP - N
    epad = EP - EMB
    A_cp = jnp.pad(A, ((0, 0), (0, pad)))
    A_hi = A_cp.astype(jnp.bfloat16)
    A_lo = (A_cp - A_hi.astype(jnp.float32)).astype(jnp.bfloat16)
    A_p = jnp.pad(A, ((0, pad), (0, pad)))
    ib_p = jnp.pad(I_B, (0, pad)).reshape(1, NP)
    W_r = jnp.pad(jax.nn.relu(I_B)[:, None] * W_enc, ((0, 0), (0, epad)))
    W_r_hi = W_r.astype(jnp.bfloat16)
    W_r_lo = (W_r - W_r_hi.astype(jnp.float32)).astype(jnp.bfloat16)
    W_enc_p = jnp.pad(W_enc, ((0, pad), (0, epad)))
    benc = jnp.pad(b_enc, (0, epad)).reshape(1, EP)
    Wx_p = jnp.pad(Wx, ((0, epad), (0, 0)))
    lenf = (bseq_length.astype(jnp.float32) - 1.0).reshape(B, 1)
    cb = C_B.reshape(1, 1)
    bl = b_lstm.reshape(1, 4 * U)
    W_H_p = jnp.pad(W_H, ((0, 0), (0, pad)))

    whole = lambda *_: (0, 0)
    embv = pl.pallas_call(
        _enc_body,
        grid=(B // BE,),
        in_specs=[
            pl.BlockSpec((BE, L, N), lambda i: (i, 0, 0)),
            pl.BlockSpec((N, NP), lambda i: (0, 0)),
            pl.BlockSpec((N, NP), lambda i: (0, 0)),
            pl.BlockSpec((1, 1), lambda i: (0, 0)),
            pl.BlockSpec((N, EP), lambda i: (0, 0)),
            pl.BlockSpec((N, EP), lambda i: (0, 0)),
            pl.BlockSpec((NP, EP), lambda i: (0, 0)),
            pl.BlockSpec((1, EP), lambda i: (0, 0)),
        ],
        out_specs=pl.BlockSpec((BE, L * EP), lambda i: (i, 0)),
        out_shape=jax.ShapeDtypeStruct((B, L * EP), jnp.float32),
        compiler_params=pltpu.CompilerParams(
            vmem_limit_bytes=120 * 1024 * 1024),
        interpret=interpret,
    )(bseq, A_hi, A_lo, cb, W_r_hi, W_r_lo, W_enc_p, benc)

    out = pl.pallas_call(
        _lstm_body,
        grid=(B // BB, L),
        in_specs=[
            pl.BlockSpec((BB, EP), lambda i, t: (i, t)),
            pl.BlockSpec((BB, 1), lambda i, t: (i, 0)),
            pl.BlockSpec((NP, NP), whole),
            pl.BlockSpec((1, NP), whole),
            pl.BlockSpec((EP, 4 * U), whole),
            pl.BlockSpec((U, 4 * U), whole),
            pl.BlockSpec((1, 4 * U), whole),
            pl.BlockSpec((U, NP), whole),
        ],
        out_specs=pl.BlockSpec((BB, N), lambda i, t: (i, 0)),
        out_shape=jax.ShapeDtypeStruct((B, N), jnp.float32),
        scratch_shapes=[
            pltpu.VMEM((BB, U), jnp.float32),
            pltpu.VMEM((BB, U), jnp.float32),
            pltpu.VMEM((BB, U), jnp.float32),
        ],
        interpret=interpret,
    )(embv, lenf, A_p, ib_p, Wx_p, Wh, bl, W_H_p)
    return out


# speedup vs baseline: 1.6948x; 1.6948x over previous
import functools

import jax
import jax.numpy as jnp
from jax.experimental import pallas as pl
from jax.experimental.pallas import tpu as pltpu

B, L, N = 1024, 20, 1000
EMB, U = 64, 128
ALPHA = 0.5
NP = 1024
EP = 128
BE = 256
BB = 256


def _enc_body(x_ref, a_ref, cb_ref, wr_ref, wenc_ref, benc_ref, emb_ref):
    thr = jnp.abs(cb_ref[0, 0])
    for t in range(L):
        x = x_ref[:, t, :]
        xa = jnp.dot(x, a_ref[...], preferred_element_type=jnp.float32)
        ga = jax.nn.relu(xa - thr)
        emb = jnp.dot(x, wr_ref[...], preferred_element_type=jnp.float32)
        emb += jnp.dot(ga, wenc_ref[...], preferred_element_type=jnp.float32)
        emb_ref[:, t * EP:(t + 1) * EP] = jax.nn.relu(emb + benc_ref[...])


def _lstm_body(emb_ref, lenf_ref, a_ref, ib_ref, wcat_ref, bl_ref,
               wH_ref, out_ref, h_ref, c_ref, hT_ref):
    t = pl.program_id(1)

    @pl.when(t == 0)
    def _init():
        h_ref[...] = jnp.zeros_like(h_ref)
        c_ref[...] = jnp.zeros_like(c_ref)
        hT_ref[...] = jnp.zeros_like(hT_ref)

    z = jnp.dot(jnp.concatenate([emb_ref[...], h_ref[...]], axis=1),
                wcat_ref[...], preferred_element_type=jnp.float32)
    z += bl_ref[...]
    i = jax.nn.sigmoid(z[:, :U])
    f = jax.nn.sigmoid(z[:, U:2 * U])
    g = jnp.tanh(z[:, 2 * U:3 * U])
    o = jax.nn.sigmoid(z[:, 3 * U:])
    c = f * c_ref[...] + i * g
    h = o * jnp.tanh(c)
    c_ref[...] = c
    h_ref[...] = h
    mask = lenf_ref[...] == t.astype(jnp.float32)
    hT_ref[...] = jnp.where(mask, h, hT_ref[...])

    @pl.when(t == L - 1)
    def _head():
        hT = hT_ref[...]
        p = jax.nn.sigmoid(
            jnp.dot(hT, wH_ref[...], preferred_element_type=jnp.float32))
        pa = jnp.dot(p, a_ref[...], preferred_element_type=jnp.float32)
        r = jax.nn.relu(ib_ref[...])
        logits = (1.0 - ALPHA) * p + ALPHA * (p * r + jax.nn.relu(pa))
        out_ref[...] = jax.nn.sigmoid(logits)[:, :N]


@functools.partial(jax.jit, static_argnames=("interpret",))
def kernel(bseq, bseq_length, A, I_B, C_B, W_enc, b_enc, Wx, Wh, b_lstm, W_H,
           interpret=False):
    pad = NP - N
    epad = EP - EMB
    A_cp = jnp.pad(A, ((0, 0), (0, pad)))
    A_p = jnp.pad(A, ((0, pad), (0, pad)))
    ib_p = jnp.pad(I_B, (0, pad)).reshape(1, NP)
    W_r = jnp.pad(jax.nn.relu(I_B)[:, None] * W_enc, ((0, 0), (0, epad)))
    W_enc_p = jnp.pad(W_enc, ((0, pad), (0, epad)))
    benc = jnp.pad(b_enc, (0, epad)).reshape(1, EP)
    W_cat = jnp.concatenate([jnp.pad(Wx, ((0, epad), (0, 0))), Wh], axis=0)
    lenf = (bseq_length.astype(jnp.float32) - 1.0).reshape(B, 1)
    cb = C_B.reshape(1, 1)
    bl = b_lstm.reshape(1, 4 * U)
    W_H_p = jnp.pad(W_H, ((0, 0), (0, pad)))

    whole = lambda *_: (0, 0)
    embv = pl.pallas_call(
        _enc_body,
        grid=(B // BE,),
        in_specs=[
            pl.BlockSpec((BE, L, N), lambda i: (i, 0, 0)),
            pl.BlockSpec((N, NP), lambda i: (0, 0)),
            pl.BlockSpec((1, 1), lambda i: (0, 0)),
            pl.BlockSpec((N, EP), lambda i: (0, 0)),
            pl.BlockSpec((NP, EP), lambda i: (0, 0)),
            pl.BlockSpec((1, EP), lambda i: (0, 0)),
        ],
        out_specs=pl.BlockSpec((BE, L * EP), lambda i: (i, 0)),
        out_shape=jax.ShapeDtypeStruct((B, L * EP), jnp.float32),
        compiler_params=pltpu.CompilerParams(
            vmem_limit_bytes=63 * 1024 * 1024),
        interpret=interpret,
    )(bseq, A_cp, cb, W_r, W_enc_p, benc)

    out = pl.pallas_call(
        _lstm_body,
        grid=(B // BB, L),
        in_specs=[
            pl.BlockSpec((BB, EP), lambda i, t: (i, t)),
            pl.BlockSpec((BB, 1), lambda i, t: (i, 0)),
            pl.BlockSpec((NP, NP), whole),
            pl.BlockSpec((1, NP), whole),
            pl.BlockSpec((2 * EP, 4 * U), whole),
            pl.BlockSpec((1, 4 * U), whole),
            pl.BlockSpec((U, NP), whole),
        ],
        out_specs=pl.BlockSpec((BB, N), lambda i, t: (i, 0)),
        out_shape=jax.ShapeDtypeStruct((B, N), jnp.float32),
        scratch_shapes=[
            pltpu.VMEM((BB, U), jnp.float32),
            pltpu.VMEM((BB, U), jnp.float32),
            pltpu.VMEM((BB, U), jnp.float32),
        ],
        interpret=interpret,
    )(embv, lenf, A_p, ib_p, W_cat, bl, W_H_p)
    return out


# speedup vs baseline: 1.9263x; 1.1366x over previous
import functools

import jax
import jax.numpy as jnp
from jax.experimental import pallas as pl
from jax.experimental.pallas import tpu as pltpu

B, L, N = 1024, 20, 1000
EMB, U = 64, 128
ALPHA = 0.5
NP = 1024
EP = 128
BE = 128


def _body(x_ref, lenf_ref, a_ref, ib_ref, cb_ref, wr_ref, wenc_ref, benc_ref,
          wcat_ref, bl_ref, wH_ref, out_ref):
    thr = jnp.abs(cb_ref[0, 0])
    h = jnp.zeros((BE, U), jnp.float32)
    c = jnp.zeros((BE, U), jnp.float32)
    hT = jnp.zeros((BE, U), jnp.float32)
    for t in range(L):
        x = x_ref[:, t, :]
        xa = jnp.dot(x, a_ref[:N, :], preferred_element_type=jnp.float32)
        ga = jax.nn.relu(xa - thr)
        emb = jnp.dot(x, wr_ref[...], preferred_element_type=jnp.float32)
        emb += jnp.dot(ga, wenc_ref[...], preferred_element_type=jnp.float32)
        emb = jax.nn.relu(emb + benc_ref[...])
        z = jnp.dot(jnp.concatenate([emb, h], axis=1), wcat_ref[...],
                    preferred_element_type=jnp.float32) + bl_ref[...]
        i = jax.nn.sigmoid(z[:, :U])
        f = jax.nn.sigmoid(z[:, U:2 * U])
        g = jnp.tanh(z[:, 2 * U:3 * U])
        o = jax.nn.sigmoid(z[:, 3 * U:])
        c = f * c + i * g
        h = o * jnp.tanh(c)
        hT = jnp.where(lenf_ref[...] == float(t), h, hT)
    p = jax.nn.sigmoid(
        jnp.dot(hT, wH_ref[...], preferred_element_type=jnp.float32))
    pa = jnp.dot(p, a_ref[...], preferred_element_type=jnp.float32)
    r = jax.nn.relu(ib_ref[...])
    logits = (1.0 - ALPHA) * p + ALPHA * (p * r + jax.nn.relu(pa))
    out_ref[...] = jax.nn.sigmoid(logits)[:, :N]


@functools.partial(jax.jit, static_argnames=("interpret",))
def kernel(bseq, bseq_length, A, I_B, C_B, W_enc, b_enc, Wx, Wh, b_lstm, W_H,
           interpret=False):
    pad = NP - N
    epad = EP - EMB
    A_p = jnp.pad(A, ((0, pad), (0, pad)))
    ib_p = jnp.pad(I_B, (0, pad)).reshape(1, NP)
    W_r = jnp.pad(jax.nn.relu(I_B)[:, None] * W_enc, ((0, 0), (0, epad)))
    W_enc_p = jnp.pad(W_enc, ((0, pad), (0, epad)))
    benc = jnp.pad(b_enc, (0, epad)).reshape(1, EP)
    W_cat = jnp.concatenate([jnp.pad(Wx, ((0, epad), (0, 0))), Wh], axis=0)
    lenf = (bseq_length.astype(jnp.float32) - 1.0).reshape(B, 1)
    cb = C_B.reshape(1, 1)
    bl = b_lstm.reshape(1, 4 * U)
    W_H_p = jnp.pad(W_H, ((0, 0), (0, pad)))

    w = lambda i: (0, 0)
    out = pl.pallas_call(
        _body,
        grid=(B // BE,),
        in_specs=[
            pl.BlockSpec((BE, L, N), lambda i: (i, 0, 0)),
            pl.BlockSpec((BE, 1), lambda i: (i, 0)),
            pl.BlockSpec((NP, NP), w),
            pl.BlockSpec((1, NP), w),
            pl.BlockSpec((1, 1), w),
            pl.BlockSpec((N, EP), w),
            pl.BlockSpec((NP, EP), w),
            pl.BlockSpec((1, EP), w),
            pl.BlockSpec((2 * EP, 4 * U), w),
            pl.BlockSpec((1, 4 * U), w),
            pl.BlockSpec((U, NP), w),
        ],
        out_specs=pl.BlockSpec((BE, N), lambda i: (i, 0)),
        out_shape=jax.ShapeDtypeStruct((B, N), jnp.float32),
        compiler_params=pltpu.CompilerParams(
            vmem_limit_bytes=63 * 1024 * 1024),
        interpret=interpret,
    )(bseq, lenf, A_p, ib_p, cb, W_r, W_enc_p, benc, W_cat, bl, W_H_p)
    return out
